# baseline (speedup 1.0000x reference)
import jax
import jax.numpy as jnp
from jax import lax
from jax.experimental import pallas as pl
from jax.experimental.pallas import tpu as pltpu
from jax.experimental.pallas import tpu_sc as plsc

B, S, D = 8, 2048, 512
NC, NS, L = 2, 16, 16
NW = NC * NS
TOK_W = (B * S) // NW
CH_ROW = S // TOK_W
SUB = 32
NSUB = TOK_W // SUB
ZROW = S


def _pe_add_body(seqs_hbm, masks_hbm, pe_hbm, out_hbm,
                 mask_v, idx_v, seq_a, seq_b, pe_a, pe_b,
                 sem_sa, sem_sb, sem_pa, sem_pb, sem_oa, sem_ob):
    wid = lax.axis_index("s") * NC + lax.axis_index("c")
    row = wid // CH_ROW
    chunk = lax.rem(wid, CH_ROW)

    pltpu.sync_copy(masks_hbm.at[row], mask_v)

    def scan_step(k, run):
        m = mask_v[pl.ds(k * L, L)]
        c = plsc.cumsum(m)
        rank = run + c - 1
        idx_v[pl.ds(k * L, L)] = jnp.where(m > 0, rank, ZROW)
        return run + c[L - 1]

    lax.fori_loop(0, S // L, scan_step, jnp.int32(0))

    seq_bufs = (seq_a, seq_b)
    pe_bufs = (pe_a, pe_b)
    sem_seq = (sem_sa, sem_sb)
    sem_pe = (sem_pa, sem_pb)
    sem_out = (sem_oa, sem_ob)

    tok0 = wid * TOK_W

    def start_in(sblk):
        slot = sblk % 2
        t0 = tok0 + sblk * SUB
        pltpu.async_copy(seqs_hbm.at[pl.ds(t0, SUB)], seq_bufs[slot], sem_seq[slot])
        idx_slice = idx_v.at[pl.ds(chunk * TOK_W + sblk * SUB, SUB)]
        pltpu.async_copy(pe_hbm.at[idx_slice], pe_bufs[slot], sem_pe[slot])

    def wait_in(sblk):
        slot = sblk % 2
        t0 = tok0 + sblk * SUB
        pltpu.make_async_copy(seqs_hbm.at[pl.ds(t0, SUB)], seq_bufs[slot],
                              sem_seq[slot]).wait()
        idx_slice = idx_v.at[pl.ds(chunk * TOK_W + sblk * SUB, SUB)]
        pltpu.make_async_copy(pe_hbm.at[idx_slice], pe_bufs[slot],
                              sem_pe[slot]).wait()

    start_in(0)
    for sblk in range(NSUB):
        slot = sblk % 2
        t0 = tok0 + sblk * SUB
        if sblk + 1 < NSUB:
            if sblk >= 1:
                pt0 = tok0 + (sblk - 1) * SUB
                pltpu.make_async_copy(
                    seq_bufs[(sblk - 1) % 2], out_hbm.at[pl.ds(pt0, SUB)],
                    sem_out[(sblk - 1) % 2]).wait()
            start_in(sblk + 1)
        wait_in(sblk)

        @pl.loop(0, SUB)
        def _add(t):
            for dd in range(D // L):
                v = pe_bufs[slot][t, pl.ds(dd * L, L)]
                plsc.addupdate(seq_bufs[slot].at[t, pl.ds(dd * L, L)], v)

        pltpu.async_copy(seq_bufs[slot], out_hbm.at[pl.ds(t0, SUB)], sem_out[slot])

    for sblk in (NSUB - 2, NSUB - 1):
        slot = sblk % 2
        t0 = tok0 + sblk * SUB
        pltpu.make_async_copy(seq_bufs[slot], out_hbm.at[pl.ds(t0, SUB)],
                              sem_out[slot]).wait()


def kernel(seqs, masks, pe):
    pe_aug = jnp.concatenate([pe, jnp.zeros((1, D), jnp.float32)], axis=0)
    seqs_flat = seqs.reshape(B * S, D)
    masks_i = masks.astype(jnp.int32)
    mesh = plsc.VectorSubcoreMesh(core_axis_name="c", subcore_axis_name="s")
    out = pl.kernel(
        _pe_add_body,
        out_type=jax.ShapeDtypeStruct((B * S, D), jnp.float32),
        mesh=mesh,
        compiler_params=pltpu.CompilerParams(needs_layout_passes=False),
        scratch_types=[
            pltpu.VMEM((S,), jnp.int32),
            pltpu.VMEM((S,), jnp.int32),
            pltpu.VMEM((SUB, D), jnp.float32),
            pltpu.VMEM((SUB, D), jnp.float32),
            pltpu.VMEM((SUB, D), jnp.float32),
            pltpu.VMEM((SUB, D), jnp.float32),
            pltpu.SemaphoreType.DMA,
            pltpu.SemaphoreType.DMA,
            pltpu.SemaphoreType.DMA,
            pltpu.SemaphoreType.DMA,
            pltpu.SemaphoreType.DMA,
            pltpu.SemaphoreType.DMA,
        ],
    )(seqs_flat, masks_i, pe_aug)
    return out.reshape(B, S, D)


# speedup vs baseline: 5.1742x; 5.1742x over previous
import jax
import jax.numpy as jnp
from jax import lax
from jax.experimental import pallas as pl
from jax.experimental.pallas import tpu as pltpu
from jax.experimental.pallas import tpu_sc as plsc

B, S, D = 8, 2048, 512
NC, NS, L = 2, 16, 16
NW = NC * NS
TOK_W = (B * S) // NW
CH_ROW = S // TOK_W
SUB = 32
NSUB = TOK_W // SUB
VPT = SUB // L


def _pe_add_body(seqs_hbm, masks_hbm, pe_hbm, out_hbm,
                 mask_v, delta_v, seq_a, seq_b, pe_a, pe_b,
                 sem_sa, sem_sb, sem_pa, sem_pb, sem_oa, sem_ob):
    wid = lax.axis_index("s") * NC + lax.axis_index("c")
    row = wid // CH_ROW
    chunk = lax.rem(wid, CH_ROW)

    pltpu.sync_copy(masks_hbm.at[row], mask_v)

    def prefix_step(k, run):
        c = plsc.cumsum(mask_v[pl.ds(k * L, L)])
        return run + c[L - 1]

    run = lax.fori_loop(0, chunk * (TOK_W // L), prefix_step, jnp.int32(0))

    k0 = chunk * (TOK_W // L)
    fs_list = []
    for s in range(NSUB):
        fs_list.append(run)
        for k2 in range(VPT):
            k = k0 + s * VPT + k2
            m = mask_v[pl.ds(k * L, L)]
            c = plsc.cumsum(m)
            delta = c - 1 + (run - fs_list[s])
            delta_v[pl.ds(s * SUB + k2 * L, L)] = jnp.where(m > 0, delta, SUB)
            run = run + c[L - 1]

    seq_bufs = (seq_a, seq_b)
    pe_bufs = (pe_a, pe_b)
    sem_seq = (sem_sa, sem_sb)
    sem_pe = (sem_pa, sem_pb)
    sem_out = (sem_oa, sem_ob)

    tok0 = wid * TOK_W

    def start_in(sblk):
        slot = sblk % 2
        t0 = tok0 + sblk * SUB
        pltpu.async_copy(seqs_hbm.at[pl.ds(t0, SUB)], seq_bufs[slot], sem_seq[slot])
        pltpu.async_copy(pe_hbm.at[pl.ds(fs_list[sblk] * D, SUB * D)],
                         pe_bufs[slot], sem_pe[slot])

    def wait_in(sblk):
        slot = sblk % 2
        t0 = tok0 + sblk * SUB
        pltpu.make_async_copy(seqs_hbm.at[pl.ds(t0, SUB)], seq_bufs[slot],
                              sem_seq[slot]).wait()
        pltpu.make_async_copy(pe_hbm.at[pl.ds(fs_list[sblk] * D, SUB * D)],
                              pe_bufs[slot], sem_pe[slot]).wait()

    start_in(0)
    for sblk in range(NSUB):
        slot = sblk % 2
        t0 = tok0 + sblk * SUB
        if sblk + 1 < NSUB:
            if sblk >= 1:
                pt0 = tok0 + (sblk - 1) * SUB
                pltpu.make_async_copy(
                    seq_bufs[(sblk - 1) % 2], out_hbm.at[pl.ds(pt0, SUB)],
                    sem_out[(sblk - 1) % 2]).wait()
            start_in(sblk + 1)
        wait_in(sblk)

        @pl.loop(0, SUB)
        def _add(t):
            d = delta_v[pl.ds(sblk * SUB + t, L)][0]

            @pl.when(d < SUB)
            def _():
                for dd in range(D // L):
                    v = pe_bufs[slot][pl.ds(d * D + dd * L, L)]
                    plsc.addupdate(seq_bufs[slot].at[t, pl.ds(dd * L, L)], v)

        pltpu.async_copy(seq_bufs[slot], out_hbm.at[pl.ds(t0, SUB)], sem_out[slot])

    for sblk in (NSUB - 2, NSUB - 1):
        slot = sblk % 2
        t0 = tok0 + sblk * SUB
        pltpu.make_async_copy(seq_bufs[slot], out_hbm.at[pl.ds(t0, SUB)],
                              sem_out[slot]).wait()


def kernel(seqs, masks, pe):
    seqs_flat = seqs.reshape(B * S, D)
    masks_i = masks.astype(jnp.int32)
    mesh = plsc.VectorSubcoreMesh(core_axis_name="c", subcore_axis_name="s")
    out = pl.kernel(
        _pe_add_body,
        out_type=jax.ShapeDtypeStruct((B * S, D), jnp.float32),
        mesh=mesh,
        compiler_params=pltpu.CompilerParams(needs_layout_passes=False),
        scratch_types=[
            pltpu.VMEM((S,), jnp.int32),
            pltpu.VMEM((TOK_W + L,), jnp.int32),
            pltpu.VMEM((SUB, D), jnp.float32),
            pltpu.VMEM((SUB, D), jnp.float32),
            pltpu.VMEM((SUB * D,), jnp.float32),
            pltpu.VMEM((SUB * D,), jnp.float32),
            pltpu.SemaphoreType.DMA,
            pltpu.SemaphoreType.DMA,
            pltpu.SemaphoreType.DMA,
            pltpu.SemaphoreType.DMA,
            pltpu.SemaphoreType.DMA,
            pltpu.SemaphoreType.DMA,
        ],
    )(seqs_flat, masks_i, pe.reshape(S * D))
    return out.reshape(B, S, D)
